# Initial kernel scaffold; baseline (speedup 1.0000x reference)
import jax
import jax.numpy as jnp
from jax.experimental import pallas as pl


def kernel(x, edge_index, P, ep, W, b):
    raise NotImplementedError("write your pallas kernel here")


# speedup vs baseline: 10.4915x; 10.4915x over previous
import functools

import jax
import jax.numpy as jnp
from jax import lax
from jax.experimental import pallas as pl
from jax.experimental.pallas import tpu as pltpu
from jax.experimental.pallas import tpu_sc as plsc

_LANES = 16
_CHUNK = 128


def _sc_segment_accumulate(x, src3, dst3, ep3, n_nodes, n_pad_rows, n_chunks):
  d = x.shape[1]
  n_sub = 16
  rows_per_tile_pad = n_pad_rows // n_sub
  mesh = plsc.VectorSubcoreMesh(core_axis_name="c", subcore_axis_name="s")

  @functools.partial(
      pl.kernel,
      mesh=mesh,
      out_type=jax.ShapeDtypeStruct((2, n_pad_rows, d), jnp.float32),
      scratch_types=[
          pltpu.VMEM((n_chunks, _CHUNK), jnp.int32),
          pltpu.VMEM((n_chunks, _CHUNK), jnp.int32),
          pltpu.VMEM((n_chunks * _CHUNK,), jnp.float32),
          pltpu.VMEM((_CHUNK, d), jnp.float32),
          pltpu.VMEM_SHARED((n_pad_rows, d), jnp.float32),
          pltpu.SemaphoreType.DMA,
      ],
  )
  def sc_kern(x_hbm, src_hbm, dst_hbm, ep_hbm, out_hbm,
              src_v, dst_v, ep_v, rows_v, ans_sh, sem):
    c = lax.axis_index("c")
    s = lax.axis_index("s")
    tile = c * n_sub + s

    zero16 = jnp.zeros((_LANES,), jnp.float32)

    def zero_rows(i, carry):
      for j in range(d // _LANES):
        rows_v[i, pl.ds(j * _LANES, _LANES)] = zero16
      return carry

    lax.fori_loop(0, _CHUNK, zero_rows, 0)

    zbase = s * rows_per_tile_pad
    for k in range(rows_per_tile_pad // _CHUNK):
      pltpu.sync_copy(rows_v, ans_sh.at[pl.ds(zbase + k * _CHUNK, _CHUNK)])

    pltpu.sync_copy(src_hbm.at[tile], src_v)
    pltpu.sync_copy(dst_hbm.at[tile], dst_v)
    pltpu.sync_copy(ep_hbm.at[tile], ep_v)

    one16 = jnp.ones((_LANES,), jnp.float32)

    def inv_body(i, carry):
      sl = pl.ds(i * _LANES, _LANES)
      ep_v[sl] = one16 / ep_v[sl]
      return carry

    lax.fori_loop(0, n_chunks * _CHUNK // _LANES, inv_body, 0)

    plsc.subcore_barrier()

    dnums = lax.GatherDimensionNumbers(
        offset_dims=(), collapsed_slice_dims=(0,), start_index_map=(0,))

    def chunk_body(ci, carry):
      pltpu.async_copy(x_hbm.at[src_v.at[ci]], rows_v, sem).wait()

      def group_body(g, gcarry):
        start = pl.multiple_of((ci * (_CHUNK // _LANES) + g) * _LANES, _LANES)
        sv16 = ep_v[pl.ds(start, _LANES)]
        row0 = g * _LANES
        for k in range(_LANES):
          idx = jnp.full((_LANES, 1), k, jnp.int32)
          sk = lax.gather(sv16, idx, dnums, (1,),
                          mode=lax.GatherScatterMode.PROMISE_IN_BOUNDS)
          for j in range(d // _LANES):
            sl = pl.ds(j * _LANES, _LANES)
            rows_v[row0 + k, sl] = rows_v[row0 + k, sl] * sk
        return gcarry

      lax.fori_loop(0, _CHUNK // _LANES, group_body, 0)
      pltpu.sync_copy(rows_v, ans_sh.at[dst_v.at[ci]], add=True)
      return carry

    lax.fori_loop(0, n_chunks, chunk_body, 0)

    plsc.subcore_barrier()

    obase = s * rows_per_tile_pad
    for k in range(rows_per_tile_pad // _CHUNK):
      pltpu.sync_copy(ans_sh.at[pl.ds(obase + k * _CHUNK, _CHUNK)],
                      out_hbm.at[c, pl.ds(obase + k * _CHUNK, _CHUNK)])

  return sc_kern(x, src3, dst3, ep3)


def _tc_dense(part0, part1, p_col, w, b_row):
  n = p_col.shape[0]
  d = part0.shape[1]
  dim = w.shape[1]
  block = 1000
  grid = n // block

  def body(p0_ref, p1_ref, p_ref, w_ref, b_ref, o_ref):
    ans = (p0_ref[...] + p1_ref[...]) * p_ref[...]
    acc = jnp.dot(ans, w_ref[...], preferred_element_type=jnp.float32)
    o_ref[...] = jnp.tanh(acc + b_ref[...])

  return pl.pallas_call(
      body,
      grid=(grid,),
      in_specs=[
          pl.BlockSpec((block, d), lambda i: (i, 0)),
          pl.BlockSpec((block, d), lambda i: (i, 0)),
          pl.BlockSpec((block, 1), lambda i: (i, 0)),
          pl.BlockSpec((d, dim), lambda i: (0, 0)),
          pl.BlockSpec((1, dim), lambda i: (0, 0)),
      ],
      out_specs=pl.BlockSpec((block, dim), lambda i: (i, 0)),
      out_shape=jax.ShapeDtypeStruct((n, dim), jnp.float32),
  )(part0, part1, p_col, w, b_row)


def kernel(x, edge_index, P, ep, W, b):
  n, d = x.shape
  e = edge_index.shape[1]
  n_tiles = 32
  n_chunks = -(-e // (n_tiles * _CHUNK))
  e_pad = n_tiles * n_chunks * _CHUNK
  pad = e_pad - e
  n_pad_rows = -(-n // (16 * _CHUNK)) * (16 * _CHUNK)

  src = edge_index[0].astype(jnp.int32)
  dst = edge_index[1].astype(jnp.int32)
  ep_f = ep.astype(jnp.float32)
  if pad:
    src = jnp.concatenate([src, jnp.zeros((pad,), jnp.int32)])
    dst = jnp.concatenate([dst, jnp.full((pad,), n, jnp.int32)])
    ep_f = jnp.concatenate([ep_f, jnp.ones((pad,), jnp.float32)])

  src3 = src.reshape(n_tiles, n_chunks, _CHUNK)
  dst3 = dst.reshape(n_tiles, n_chunks, _CHUNK)
  ep3 = ep_f.reshape(n_tiles, n_chunks * _CHUNK)

  parts = _sc_segment_accumulate(x, src3, dst3, ep3, n, n_pad_rows, n_chunks)
  return _tc_dense(parts[0], parts[1], P[:, None], W, b[None, :])


# speedup vs baseline: 15.1078x; 1.4400x over previous
import functools

import jax
import jax.numpy as jnp
from jax import lax
from jax.experimental import pallas as pl
from jax.experimental.pallas import tpu as pltpu
from jax.experimental.pallas import tpu_sc as plsc

_LANES = 16
_CHUNK = 112


def _sc_segment_accumulate(x, src3, dst3, ep3, n_nodes, n_pad_rows, n_chunks):
  d = x.shape[1]
  n_sub = 16
  rows_per_tile_pad = n_pad_rows // n_sub
  mesh = plsc.VectorSubcoreMesh(core_axis_name="c", subcore_axis_name="s")

  @functools.partial(
      pl.kernel,
      mesh=mesh,
      out_type=jax.ShapeDtypeStruct((2, n_pad_rows, d), jnp.float32),
      scratch_types=[
          pltpu.VMEM((3, _CHUNK), jnp.int32),
          pltpu.VMEM((3, _CHUNK), jnp.int32),
          pltpu.VMEM((3, _CHUNK), jnp.float32),
          pltpu.VMEM((_CHUNK, d), jnp.float32),
          pltpu.VMEM((_CHUNK, d), jnp.float32),
          pltpu.VMEM_SHARED((n_pad_rows, d), jnp.float32),
          pltpu.SemaphoreType.DMA,
          pltpu.SemaphoreType.DMA,
          pltpu.SemaphoreType.DMA,
          pltpu.SemaphoreType.DMA,
          pltpu.SemaphoreType.DMA,
      ],
  )
  def sc_kern(x_hbm, src_hbm, dst_hbm, ep_hbm, out_hbm,
              src_r, dst_r, ep_r, r0_v, r1_v, ans_sh,
              gsem0, gsem1, msem0, msem1, msem2):
    rbuf = (r0_v, r1_v)
    gsem = (gsem0, gsem1)
    msem = (msem0, msem1, msem2)
    rows_v = r0_v
    c = lax.axis_index("c")
    s = lax.axis_index("s")
    tile = c * n_sub + s

    zero16 = jnp.zeros((_LANES,), jnp.float32)

    def zero_rows(i, carry):
      for j in range(d // _LANES):
        rows_v[i, pl.ds(j * _LANES, _LANES)] = zero16
      return carry

    lax.fori_loop(0, _CHUNK, zero_rows, 0)

    zbase = s * rows_per_tile_pad
    for k in range(rows_per_tile_pad // _CHUNK):
      pltpu.sync_copy(rows_v, ans_sh.at[pl.ds(zbase + k * _CHUNK, _CHUNK)])

    plsc.subcore_barrier()

    def fetch_meta(ci, m, sem):
      pltpu.async_copy(src_hbm.at[tile, ci], src_r.at[m], sem)
      pltpu.async_copy(dst_hbm.at[tile, ci], dst_r.at[m], sem)
      pltpu.async_copy(ep_hbm.at[tile, ci], ep_r.at[m], sem)

    def wait_meta(ci, m, sem):
      pltpu.make_async_copy(src_hbm.at[tile, ci], src_r.at[m], sem).wait()
      pltpu.make_async_copy(dst_hbm.at[tile, ci], dst_r.at[m], sem).wait()
      pltpu.make_async_copy(ep_hbm.at[tile, ci], ep_r.at[m], sem).wait()

    fetch_meta(0, 0, msem[0])
    fetch_meta(1, 1, msem[1])
    wait_meta(0, 0, msem[0])
    pltpu.async_copy(x_hbm.at[src_r.at[0]], rbuf[0], gsem[0])

    dnums = lax.GatherDimensionNumbers(
        offset_dims=(), collapsed_slice_dims=(0,), start_index_map=(0,))

    def scale_chunk(m, buf):
      def group_body(g, gcarry):
        start = pl.multiple_of(g * _LANES, _LANES)
        sv16 = ep_r[m, pl.ds(start, _LANES)]
        row0 = g * _LANES
        for k in range(_LANES):
          idx = jnp.full((_LANES, 1), k, jnp.int32)
          sk = lax.gather(sv16, idx, dnums, (1,),
                          mode=lax.GatherScatterMode.PROMISE_IN_BOUNDS)
          for j in range(d // _LANES):
            sl = pl.ds(j * _LANES, _LANES)
            buf[row0 + k, sl] = buf[row0 + k, sl] * sk
        return gcarry

      lax.fori_loop(0, _CHUNK // _LANES, group_body, 0)

    one16 = jnp.ones((_LANES,), jnp.float32)

    def outer_body(i, carry):
      for u in range(6):
        ci = i * 6 + u
        b = u % 2
        o = 1 - b
        m = u % 3

        pltpu.make_async_copy(x_hbm.at[src_r.at[m]], rbuf[b], gsem[b]).wait()

        @pl.when(ci + 2 < n_chunks)
        def _fetch_meta():
          fetch_meta(ci + 2, (m + 2) % 3, msem[(m + 2) % 3])

        @pl.when(ci + 1 < n_chunks)
        def _next_gather():
          wait_meta(ci + 1, (m + 1) % 3, msem[(m + 1) % 3])
          pltpu.async_copy(x_hbm.at[src_r.at[(m + 1) % 3]], rbuf[o], gsem[o])

        for g in range(_CHUNK // _LANES):
          sl = pl.ds(g * _LANES, _LANES)
          ep_r[m, sl] = one16 / ep_r[m, sl]

        scale_chunk(m, rbuf[b])
        pltpu.sync_copy(rbuf[b], ans_sh.at[dst_r.at[m]], add=True)

      return carry

    lax.fori_loop(0, n_chunks // 6, outer_body, 0)

    plsc.subcore_barrier()

    obase = s * rows_per_tile_pad
    for k in range(rows_per_tile_pad // _CHUNK):
      pltpu.sync_copy(ans_sh.at[pl.ds(obase + k * _CHUNK, _CHUNK)],
                      out_hbm.at[c, pl.ds(obase + k * _CHUNK, _CHUNK)])

  return sc_kern(x, src3, dst3, ep3)


def _tc_dense(part0, part1, p_col, w, b_row):
  n = p_col.shape[0]
  d = part0.shape[1]
  dim = w.shape[1]
  block = 1000
  grid = n // block

  def body(p0_ref, p1_ref, p_ref, w_ref, b_ref, o_ref):
    ans = (p0_ref[...] + p1_ref[...]) * p_ref[...]
    acc = jnp.dot(ans, w_ref[...], preferred_element_type=jnp.float32)
    o_ref[...] = jnp.tanh(acc + b_ref[...])

  return pl.pallas_call(
      body,
      grid=(grid,),
      in_specs=[
          pl.BlockSpec((block, d), lambda i: (i, 0)),
          pl.BlockSpec((block, d), lambda i: (i, 0)),
          pl.BlockSpec((block, 1), lambda i: (i, 0)),
          pl.BlockSpec((d, dim), lambda i: (0, 0)),
          pl.BlockSpec((1, dim), lambda i: (0, 0)),
      ],
      out_specs=pl.BlockSpec((block, dim), lambda i: (i, 0)),
      out_shape=jax.ShapeDtypeStruct((n, dim), jnp.float32),
  )(part0, part1, p_col, w, b_row)


def kernel(x, edge_index, P, ep, W, b):
  n, d = x.shape
  e = edge_index.shape[1]
  n_tiles = 32
  n_chunks = -(-e // (n_tiles * _CHUNK))
  n_chunks = -(-n_chunks // 6) * 6
  e_pad = n_tiles * n_chunks * _CHUNK
  pad = e_pad - e
  n_pad_rows = -(-n // (16 * _CHUNK)) * (16 * _CHUNK)

  src = edge_index[0].astype(jnp.int32)
  dst = edge_index[1].astype(jnp.int32)
  ep_f = ep.astype(jnp.float32)
  if pad:
    src = jnp.concatenate([src, jnp.zeros((pad,), jnp.int32)])
    dst = jnp.concatenate([dst, jnp.full((pad,), n, jnp.int32)])
    ep_f = jnp.concatenate([ep_f, jnp.ones((pad,), jnp.float32)])

  src3 = src.reshape(n_tiles, n_chunks, _CHUNK)
  dst3 = dst.reshape(n_tiles, n_chunks, _CHUNK)
  ep3 = ep_f.reshape(n_tiles, n_chunks, _CHUNK)

  parts = _sc_segment_accumulate(x, src3, dst3, ep3, n, n_pad_rows, n_chunks)
  return _tc_dense(parts[0], parts[1], P[:, None], W, b[None, :])
